# baseline (speedup 1.0000x reference)
import jax
import jax.numpy as jnp
from jax import lax
from jax.experimental import pallas as pl
from jax.experimental.pallas import tpu as pltpu
from jax.experimental.pallas import tpu_sc as plsc

B, L, H = 4096, 50, 768
N = B * L
NC, NS = 2, 16
NW = NC * NS
TOK_PER_W = N // NW
T = 64
NCHUNK = TOK_PER_W // T
LANES = 16
HV = H // LANES


def _body(emb_hbm, ids_hbm, table_hbm, out_hbm, idx_v, buf_e, buf_g, sem_e, sem_g):
    wid = lax.axis_index("s") * NC + lax.axis_index("c")
    base = wid * TOK_PER_W

    def chunk(c, carry):
        off = base + c * T
        pltpu.sync_copy(ids_hbm.at[pl.ds(off, T)], idx_v)
        cp_e = pltpu.async_copy(emb_hbm.at[pl.ds(off, T)], buf_e, sem_e)
        cp_g = pltpu.async_copy(table_hbm.at[idx_v], buf_g, sem_g)
        cp_e.wait()
        cp_g.wait()

        def row(r, carry2):
            for j in range(HV):
                plsc.addupdate(buf_e.at[r, pl.ds(j * LANES, LANES)],
                               buf_g[r, pl.ds(j * LANES, LANES)])
            return carry2

        lax.fori_loop(0, T, row, 0)
        pltpu.sync_copy(buf_e, out_hbm.at[pl.ds(off, T)])
        return carry

    lax.fori_loop(0, NCHUNK, chunk, 0)


@jax.jit
def kernel(batch_Phrase_emb, Phrase_type_ids, phrase_attribute_emb_all):
    emb = batch_Phrase_emb.reshape(N, H)
    ids = Phrase_type_ids.reshape(N).astype(jnp.int32)

    run = pl.kernel(
        _body,
        out_type=jax.ShapeDtypeStruct((N, H), jnp.float32),
        mesh=plsc.VectorSubcoreMesh(core_axis_name="c", subcore_axis_name="s"),
        scratch_types=[
            pltpu.VMEM((T,), jnp.int32),
            pltpu.VMEM((T, H), jnp.float32),
            pltpu.VMEM((T, H), jnp.float32),
            pltpu.SemaphoreType.DMA,
            pltpu.SemaphoreType.DMA,
        ],
    )
    out = run(emb, ids, phrase_attribute_emb_all)
    return out.reshape(B, L, H)


# speedup vs baseline: 1.0500x; 1.0500x over previous
import jax
import jax.numpy as jnp
from jax import lax
from jax.experimental import pallas as pl
from jax.experimental.pallas import tpu as pltpu
from jax.experimental.pallas import tpu_sc as plsc

B, L, H = 4096, 50, 768
N = B * L
NC, NS = 2, 16
NW = NC * NS
TOK_PER_W = N // NW
T = 32
NCHUNK = TOK_PER_W // T
LANES = 16
HV = H // LANES


def _body(emb_hbm, ids_hbm, table_hbm, out_hbm,
          idx_all, buf_e, buf_g, sem_e, sem_g, sem_o):
    wid = lax.axis_index("s") * NC + lax.axis_index("c")
    base = wid * TOK_PER_W

    def start_in(c, b):
        off = base + c * T
        pltpu.async_copy(emb_hbm.at[pl.ds(off, T)], buf_e.at[b], sem_e.at[b])
        pltpu.async_copy(table_hbm.at[idx_all.at[pl.ds(c * T, T)]],
                         buf_g.at[b], sem_g.at[b])

    def wait_in(c, b):
        off = base + c * T
        pltpu.make_async_copy(emb_hbm.at[pl.ds(off, T)], buf_e.at[b],
                              sem_e.at[b]).wait()
        pltpu.make_async_copy(table_hbm.at[idx_all.at[pl.ds(c * T, T)]],
                              buf_g.at[b], sem_g.at[b]).wait()

    def start_out(c, b):
        off = base + c * T
        pltpu.async_copy(buf_e.at[b], out_hbm.at[pl.ds(off, T)], sem_o.at[b])

    def wait_out(c, b):
        off = base + c * T
        pltpu.make_async_copy(buf_e.at[b], out_hbm.at[pl.ds(off, T)],
                              sem_o.at[b]).wait()

    pltpu.sync_copy(ids_hbm.at[pl.ds(base, TOK_PER_W)], idx_all)

    start_in(0, 0)
    start_in(1, 1)

    def outer(c2, carry):
        for b in range(2):
            c = c2 * 2 + b
            wait_in(c, b)

            def row(r, carry2):
                for j in range(HV):
                    plsc.addupdate(buf_e.at[b, r, pl.ds(j * LANES, LANES)],
                                   buf_g[b, r, pl.ds(j * LANES, LANES)])
                return carry2

            lax.fori_loop(0, T, row, 0)
            start_out(c, b)

            @pl.when(c + 2 < NCHUNK)
            def _prefetch():
                wait_out(c, b)
                start_in(c + 2, b)
        return carry

    lax.fori_loop(0, NCHUNK // 2, outer, 0)
    wait_out(NCHUNK - 2, 0)
    wait_out(NCHUNK - 1, 1)


@jax.jit
def kernel(batch_Phrase_emb, Phrase_type_ids, phrase_attribute_emb_all):
    emb = batch_Phrase_emb.reshape(N, H)
    ids = Phrase_type_ids.reshape(N).astype(jnp.int32)

    run = pl.kernel(
        _body,
        out_type=jax.ShapeDtypeStruct((N, H), jnp.float32),
        mesh=plsc.VectorSubcoreMesh(core_axis_name="c", subcore_axis_name="s"),
        scratch_types=[
            pltpu.VMEM((TOK_PER_W,), jnp.int32),
            pltpu.VMEM((2, T, H), jnp.float32),
            pltpu.VMEM((2, T, H), jnp.float32),
            pltpu.SemaphoreType.DMA((2,)),
            pltpu.SemaphoreType.DMA((2,)),
            pltpu.SemaphoreType.DMA((2,)),
        ],
    )
    out = run(emb, ids, phrase_attribute_emb_all)
    return out.reshape(B, L, H)
